# baseline (speedup 1.0000x reference)
import functools

import jax
import jax.numpy as jnp
from jax import lax
from jax.experimental import pallas as pl
from jax.experimental.pallas import tpu as pltpu
from jax.experimental.pallas import tpu_sc as plsc

_N = 10000
_E = 320000
_D = 128
_EPS = 1e-5
_L = 16
_NCH = _D // _L
_NH = _N // 2
_NHP = 5008
_EH = _E // 2
_B = 800
_NBLK = _EH // _B
_GCH = 128

_NP = 10016
_DUMMY_G = _N * _NCH
_DUMMY_A = _NH

_ROWS_TC = 2504
_ROWS_TC2 = 2000


def _tc_pool(h, W, b):
  def body(h_ref, w_ref, b_ref, o_ref):
    i = pl.program_id(0)
    y = lax.dot_general(h_ref[...], w_ref[...], (((1,), (1,)), ((), ())),
                        preferred_element_type=jnp.float32,
                        precision=lax.Precision.HIGHEST)
    y = y + b_ref[...]
    row = i * _ROWS_TC + lax.broadcasted_iota(jnp.int32, y.shape, 0)
    o_ref[...] = jnp.where(row < _N, jnp.maximum(y, 0.0), 0.0)

  return pl.pallas_call(
      body,
      out_shape=jax.ShapeDtypeStruct((_NP, _D), jnp.float32),
      grid=(_NP // _ROWS_TC,),
      in_specs=[
          pl.BlockSpec((_ROWS_TC, _D), lambda i: (i, 0)),
          pl.BlockSpec((_D, _D), lambda i: (0, 0)),
          pl.BlockSpec((1, _D), lambda i: (0, 0)),
      ],
      out_specs=pl.BlockSpec((_ROWS_TC, _D), lambda i: (i, 0)),
  )(h, W, b.reshape(1, _D))


def _tc_combine(h, agg, Ws, Wn, bias, g, be):
  def body(h_ref, a_ref, ws_ref, wn_ref, b_ref, g_ref, e_ref, o_ref):
    x = lax.dot_general(h_ref[...], ws_ref[...], (((1,), (1,)), ((), ())),
                        preferred_element_type=jnp.float32,
                        precision=lax.Precision.HIGHEST)
    x = x + lax.dot_general(a_ref[...], wn_ref[...], (((1,), (1,)), ((), ())),
                            preferred_element_type=jnp.float32,
                            precision=lax.Precision.HIGHEST)
    x = x + b_ref[...]
    mu = jnp.mean(x, axis=1, keepdims=True)
    xc = x - mu
    var = jnp.mean(xc * xc, axis=1, keepdims=True)
    y = xc * lax.rsqrt(var + _EPS) * g_ref[...] + e_ref[...]
    o_ref[...] = jnp.maximum(y, 0.0)

  return pl.pallas_call(
      body,
      out_shape=jax.ShapeDtypeStruct((_N, _D), jnp.float32),
      grid=(_N // _ROWS_TC2,),
      in_specs=[
          pl.BlockSpec((_ROWS_TC2, _D), lambda i: (i, 0)),
          pl.BlockSpec((_ROWS_TC2, _D), lambda i: (i, 0)),
          pl.BlockSpec((_D, _D), lambda i: (0, 0)),
          pl.BlockSpec((_D, _D), lambda i: (0, 0)),
          pl.BlockSpec((1, _D), lambda i: (0, 0)),
          pl.BlockSpec((1, _D), lambda i: (0, 0)),
          pl.BlockSpec((1, _D), lambda i: (0, 0)),
      ],
      out_specs=pl.BlockSpec((_ROWS_TC2, _D), lambda i: (i, 0)),
  )(h, agg, Ws, Wn, bias.reshape(1, _D), g.reshape(1, _D), be.reshape(1, _D))


def _sc_segment_max(hp_flat, edge_index):
  mesh = plsc.VectorSubcoreMesh(core_axis_name="c", subcore_axis_name="s")

  @functools.partial(
      pl.kernel,
      out_type=(jax.ShapeDtypeStruct((_NCH, _N, _L), jnp.float32),
                jax.ShapeDtypeStruct((2, _NCH, _NHP, _L), jnp.float32)),
      mesh=mesh,
      compiler_params=pltpu.CompilerParams(use_tc_tiling_on_sc=False,
                                           needs_layout_passes=False),
      scratch_types=[
          pltpu.VMEM((_NHP, _L), jnp.float32),
          pltpu.VMEM((2, _B), jnp.int32),
          pltpu.VMEM((2, _B), jnp.int32),
          pltpu.VMEM((_B + _L,), jnp.int32),
          pltpu.VMEM((896,), jnp.int32),
          pltpu.VMEM((896, _L), jnp.float32),
          pltpu.VMEM((_B + _L,), jnp.int32),
          pltpu.VMEM((896,), jnp.int32),
          pltpu.VMEM((896, _L), jnp.float32),
          pltpu.SemaphoreType.DMA,
          pltpu.SemaphoreType.DMA,
          pltpu.SemaphoreType.DMA,
      ],
  )
  def k(hp_hbm, ei_hbm, out_hbm, mrg_hbm,
        agg_l, eb_a, eb_b, dloc_a, gidx_a, rows_a,
        dloc_bb, gidx_bb, rows_bb, gsem_a, gsem_b, esem):
    cid = lax.axis_index("c")
    sid = lax.axis_index("s")
    chunk = sid % _NCH
    eseg = sid // _NCH
    half = cid
    lo = half * _NH
    ebase = eseg * _EH
    lanes = lax.iota(jnp.int32, _L)

    zrow = jnp.zeros((_L,), jnp.float32)
    def zero_agg(i, carry):
      agg_l[i] = zrow
      return carry
    lax.fori_loop(0, _NHP, zero_agg, 0)

    zidx = jnp.zeros((_L,), jnp.int32)
    def mk_zero_gidx(gidxb):
      def zero_gidx(i, carry):
        gidxb[pl.ds(i * _L, _L)] = zidx
        return carry
      return zero_gidx
    lax.fori_loop(0, 896 // _L, mk_zero_gidx(gidx_a), 0)
    lax.fori_loop(0, 896 // _L, mk_zero_gidx(gidx_bb), 0)

    def edge_copy(b, ebuf):
      bb = lax.rem(b, _NBLK)
      base = ebase + bb * _B
      return pltpu.make_async_copy(
          ei_hbm.at[:, pl.ds(base, _B)], ebuf, esem)

    def prep_fire(b, ebuf, dlocb, gidxb, rowsb, gsem, nextbuf):
      edge_copy(b, ebuf).wait()

      def prep(i, nvec):
        d = ebuf[1, pl.ds(i * _L, _L)]
        s2 = ebuf[0, pl.ds(i * _L, _L)]
        m = (d >= lo) & (d < lo + _NH)
        cs = plsc.cumsum(jnp.where(m, 1, 0))
        tgt = nvec + cs - 1
        plsc.store_scatter(dlocb, [tgt], d - lo, mask=m)
        plsc.store_scatter(gidxb, [tgt], s2 * _NCH + chunk, mask=m)
        return nvec + plsc.all_reduce_population_count(m)
      nvec = lax.fori_loop(0, _B // _L, prep,
                           jnp.zeros((_L,), jnp.int32), unroll=4)
      plsc.store_scatter(dlocb, [nvec + lanes],
                         jnp.full((_L,), _DUMMY_A, jnp.int32))
      edge_copy(b + 2, nextbuf).start()
      nsc = nvec[0]

      nch = (nsc + _GCH - 1) // _GCH
      def fire(j, carry2):
        pltpu.make_async_copy(
            hp_hbm.at[gidxb.at[pl.ds(j * _GCH, _GCH)]],
            rowsb.at[pl.ds(j * _GCH, _GCH)],
            gsem).start()
        return carry2
      lax.fori_loop(0, nch, fire, 0)
      return nsc

    def drain_acc(dlocb, gidxb, rowsb, nsc, gsem):
      nch = (nsc + _GCH - 1) // _GCH
      def drain(j, carry2):
        pltpu.make_async_copy(
            hp_hbm.at[gidxb.at[pl.ds(j * _GCH, _GCH)]],
            rowsb.at[pl.ds(j * _GCH, _GCH)],
            gsem).wait()
        return carry2
      lax.fori_loop(0, nch, drain, 0)

      ngrp = (nsc + _L - 1) // _L
      def accg(gi, carry2):
        dvec = dlocb[pl.ds(gi * _L, _L)]
        erow = gi * _L + lanes
        for kk in range(_L):
          col = (lanes + kk) & (_L - 1)
          a = plsc.load_gather(agg_l, [dvec, col])
          r = plsc.load_gather(rowsb, [erow, col])
          plsc.store_scatter(agg_l, [dvec, col], jnp.maximum(a, r))
        return carry2
      lax.fori_loop(0, ngrp, accg, 0)

    edge_copy(0, eb_a).start()
    edge_copy(1, eb_b).start()
    n0 = prep_fire(0, eb_a, dloc_a, gidx_a, rows_a, gsem_a, eb_a)

    def pair(p, n_prev):
      b1 = 2 * p + 1
      n_b = prep_fire(b1, eb_b, dloc_bb, gidx_bb, rows_bb, gsem_b, eb_b)
      drain_acc(dloc_a, gidx_a, rows_a, n_prev, gsem_a)
      n_a = prep_fire(b1 + 1, eb_a, dloc_a, gidx_a, rows_a, gsem_a, eb_a)
      drain_acc(dloc_bb, gidx_bb, rows_bb, n_b, gsem_b)
      return n_a

    n_last = lax.fori_loop(0, (_NBLK - 2) // 2, pair, n0)
    n_tail = prep_fire(_NBLK - 1, eb_b, dloc_bb, gidx_bb, rows_bb, gsem_b,
                       eb_b)
    drain_acc(dloc_a, gidx_a, rows_a, n_last, gsem_a)
    drain_acc(dloc_bb, gidx_bb, rows_bb, n_tail, gsem_b)
    edge_copy(0, eb_b).wait()
    edge_copy(0, eb_a).wait()

    @pl.when(eseg == 1)
    def _publish():
      pltpu.sync_copy(agg_l.at[pl.ds(0, _NHP)], mrg_hbm.at[half, chunk])
    plsc.subcore_barrier()

    @pl.when(eseg == 0)
    def _merge():
      off = 0
      for sz in (_B, _B, _B, _B, _B, _B, _NHP - 6 * _B):
        pltpu.sync_copy(mrg_hbm.at[half, chunk, pl.ds(off, sz)],
                        rows_a.at[pl.ds(0, sz)])
        def mg(i, carry, off=off):
          r = off + i
          agg_l[r] = jnp.maximum(agg_l[r], rows_a[i])
          return carry
        lax.fori_loop(0, sz, mg, 0)
        off += sz

    @pl.when(eseg == 0)
    def _store():
      pltpu.sync_copy(agg_l.at[pl.ds(0, _NH)],
                      out_hbm.at[chunk, pl.ds(lo, _NH)])

  return k(hp_flat, edge_index)[0]


def _layer(h, edge_index, Wp, bp, Ws, Wn, bias, g, be):
  hp = _tc_pool(h, Wp, bp)
  agg3 = _sc_segment_max(hp.reshape(_NP * _NCH, _L), edge_index)
  agg = agg3.transpose(1, 0, 2).reshape(_N, _D)
  return _tc_combine(h, agg, Ws, Wn, bias, g, be)


def kernel(h, edge_index,
           W_pool0, b_pool0, W_self0, W_neigh0, bias0, ln_g0, ln_b0,
           W_pool1, b_pool1, W_self1, W_neigh1, bias1, ln_g1, ln_b1):
  h = _layer(h, edge_index, W_pool0, b_pool0, W_self0, W_neigh0, bias0, ln_g0, ln_b0)
  h = _layer(h, edge_index, W_pool1, b_pool1, W_self1, W_neigh1, bias1, ln_g1, ln_b1)
  return h


# speedup vs baseline: 2.4255x; 2.4255x over previous
import functools

import jax
import jax.numpy as jnp
from jax import lax
from jax.experimental import pallas as pl
from jax.experimental.pallas import tpu as pltpu
from jax.experimental.pallas import tpu_sc as plsc

_N = 10000
_E = 320000
_D = 128
_EPS = 1e-5
_L = 16
_NCH = _D // _L
_NH = _N // 2
_NHP = 5008
_EH = _E // 2
_B = 800
_NBLK = _EH // _B
_GCH = 80

_NP = 10016
_DUMMY_G = _N * _NCH
_DUMMY_A = _NH

_ROWS_TC = 2504
_ROWS_TC2 = 2000


def _tc_pool(h, W, b):
  def body(h_ref, w_ref, b_ref, o_ref):
    i = pl.program_id(0)
    y = lax.dot_general(h_ref[...], w_ref[...], (((1,), (1,)), ((), ())),
                        preferred_element_type=jnp.float32,
                        precision=lax.Precision.HIGHEST)
    y = y + b_ref[...]
    row = i * _ROWS_TC + lax.broadcasted_iota(jnp.int32, y.shape, 0)
    o_ref[...] = jnp.where(row < _N, jnp.maximum(y, 0.0), 0.0)

  return pl.pallas_call(
      body,
      out_shape=jax.ShapeDtypeStruct((_NP, _D), jnp.float32),
      grid=(_NP // _ROWS_TC,),
      in_specs=[
          pl.BlockSpec((_ROWS_TC, _D), lambda i: (i, 0)),
          pl.BlockSpec((_D, _D), lambda i: (0, 0)),
          pl.BlockSpec((1, _D), lambda i: (0, 0)),
      ],
      out_specs=pl.BlockSpec((_ROWS_TC, _D), lambda i: (i, 0)),
  )(h, W, b.reshape(1, _D))


def _tc_combine(h, agg, Ws, Wn, bias, g, be):
  def body(h_ref, a_ref, ws_ref, wn_ref, b_ref, g_ref, e_ref, o_ref):
    x = lax.dot_general(h_ref[...], ws_ref[...], (((1,), (1,)), ((), ())),
                        preferred_element_type=jnp.float32,
                        precision=lax.Precision.HIGHEST)
    x = x + lax.dot_general(a_ref[...], wn_ref[...], (((1,), (1,)), ((), ())),
                            preferred_element_type=jnp.float32,
                            precision=lax.Precision.HIGHEST)
    x = x + b_ref[...]
    mu = jnp.mean(x, axis=1, keepdims=True)
    xc = x - mu
    var = jnp.mean(xc * xc, axis=1, keepdims=True)
    y = xc * lax.rsqrt(var + _EPS) * g_ref[...] + e_ref[...]
    o_ref[...] = jnp.maximum(y, 0.0)

  return pl.pallas_call(
      body,
      out_shape=jax.ShapeDtypeStruct((_N, _D), jnp.float32),
      grid=(_N // _ROWS_TC2,),
      in_specs=[
          pl.BlockSpec((_ROWS_TC2, _D), lambda i: (i, 0)),
          pl.BlockSpec((_ROWS_TC2, _D), lambda i: (i, 0)),
          pl.BlockSpec((_D, _D), lambda i: (0, 0)),
          pl.BlockSpec((_D, _D), lambda i: (0, 0)),
          pl.BlockSpec((1, _D), lambda i: (0, 0)),
          pl.BlockSpec((1, _D), lambda i: (0, 0)),
          pl.BlockSpec((1, _D), lambda i: (0, 0)),
      ],
      out_specs=pl.BlockSpec((_ROWS_TC2, _D), lambda i: (i, 0)),
  )(h, agg, Ws, Wn, bias.reshape(1, _D), g.reshape(1, _D), be.reshape(1, _D))


def _sc_segment_max(hp_flat, edge_index):
  mesh = plsc.VectorSubcoreMesh(core_axis_name="c", subcore_axis_name="s")

  @functools.partial(
      pl.kernel,
      out_type=(jax.ShapeDtypeStruct((_NCH, _N, _L), jnp.float32),
                jax.ShapeDtypeStruct((2, _NCH, _NHP, _L), jnp.float32)),
      mesh=mesh,
      compiler_params=pltpu.CompilerParams(use_tc_tiling_on_sc=False,
                                           needs_layout_passes=False),
      scratch_types=[
          pltpu.VMEM((_NHP, _L), jnp.float32),
          pltpu.VMEM((2, _B), jnp.int32),
          pltpu.VMEM((2, _B), jnp.int32),
          pltpu.VMEM((_B + _L,), jnp.int32),
          pltpu.VMEM((_B,), jnp.int32),
          pltpu.VMEM((_B, _L), jnp.float32),
          pltpu.VMEM((_B + _L,), jnp.int32),
          pltpu.VMEM((_B,), jnp.int32),
          pltpu.VMEM((_B, _L), jnp.float32),
          pltpu.SemaphoreType.DMA,
          pltpu.SemaphoreType.DMA,
          pltpu.SemaphoreType.DMA,
      ],
  )
  def k(hp_hbm, ei_hbm, out_hbm, mrg_hbm,
        agg_l, eb_a, eb_b, dloc_a, gidx_a, rows_a,
        dloc_bb, gidx_bb, rows_bb, gsem_a, gsem_b, esem):
    cid = lax.axis_index("c")
    sid = lax.axis_index("s")
    chunk = sid % _NCH
    eseg = sid // _NCH
    half = cid
    lo = half * _NH
    ebase = eseg * _EH
    lanes = lax.iota(jnp.int32, _L)

    zrow = jnp.zeros((_L,), jnp.float32)
    def zero_agg(i, carry):
      agg_l[i] = zrow
      return carry
    lax.fori_loop(0, _NHP, zero_agg, 0)

    zidx = jnp.zeros((_L,), jnp.int32)
    def mk_zero_gidx(gidxb):
      def zero_gidx(i, carry):
        gidxb[pl.ds(i * _L, _L)] = zidx
        return carry
      return zero_gidx
    lax.fori_loop(0, _B // _L, mk_zero_gidx(gidx_a), 0)
    lax.fori_loop(0, _B // _L, mk_zero_gidx(gidx_bb), 0)

    def edge_copy(b, ebuf):
      bb = lax.rem(b, _NBLK)
      base = ebase + bb * _B
      return pltpu.make_async_copy(
          ei_hbm.at[:, pl.ds(base, _B)], ebuf, esem)

    def prep_fire(b, ebuf, dlocb, gidxb, rowsb, gsem, nextbuf):
      edge_copy(b, ebuf).wait()

      def prep(i, nvec):
        d = ebuf[1, pl.ds(i * _L, _L)]
        s2 = ebuf[0, pl.ds(i * _L, _L)]
        m = (d >= lo) & (d < lo + _NH)
        cs = plsc.cumsum(jnp.where(m, 1, 0))
        tgt = nvec + cs - 1
        plsc.store_scatter(dlocb, [tgt], d - lo, mask=m)
        plsc.store_scatter(gidxb, [tgt], s2 * _NCH + chunk, mask=m)
        return nvec + plsc.all_reduce_population_count(m)
      nvec = lax.fori_loop(0, _B // _L, prep,
                           jnp.zeros((_L,), jnp.int32), unroll=2)
      plsc.store_scatter(dlocb, [nvec + lanes],
                         jnp.full((_L,), _DUMMY_A, jnp.int32))
      edge_copy(b + 2, nextbuf).start()
      nsc = nvec[0]

      nch = (nsc + _GCH - 1) // _GCH
      def fire(j, carry2):
        pltpu.make_async_copy(
            hp_hbm.at[gidxb.at[pl.ds(j * _GCH, _GCH)]],
            rowsb.at[pl.ds(j * _GCH, _GCH)],
            gsem).start()
        return carry2
      lax.fori_loop(0, nch, fire, 0)
      return nsc

    def drain_acc(dlocb, gidxb, rowsb, nsc, gsem):
      nch = (nsc + _GCH - 1) // _GCH
      def drain(j, carry2):
        pltpu.make_async_copy(
            hp_hbm.at[gidxb.at[pl.ds(j * _GCH, _GCH)]],
            rowsb.at[pl.ds(j * _GCH, _GCH)],
            gsem).wait()
        return carry2
      lax.fori_loop(0, nch, drain, 0)

      ngrp = (nsc + _L - 1) // _L
      def accg(gi, carry2):
        dvec = dlocb[pl.ds(gi * _L, _L)]
        erow = gi * _L + lanes
        for kk in range(_L):
          col = (lanes + kk) & (_L - 1)
          a = plsc.load_gather(agg_l, [dvec, col])
          r = plsc.load_gather(rowsb, [erow, col])
          plsc.store_scatter(agg_l, [dvec, col], jnp.maximum(a, r))
        return carry2
      lax.fori_loop(0, ngrp, accg, 0)

    edge_copy(0, eb_a).start()
    edge_copy(1, eb_b).start()
    n0 = prep_fire(0, eb_a, dloc_a, gidx_a, rows_a, gsem_a, eb_a)

    def pair(p, n_prev):
      b1 = 2 * p + 1
      n_b = prep_fire(b1, eb_b, dloc_bb, gidx_bb, rows_bb, gsem_b, eb_b)
      drain_acc(dloc_a, gidx_a, rows_a, n_prev, gsem_a)
      n_a = prep_fire(b1 + 1, eb_a, dloc_a, gidx_a, rows_a, gsem_a, eb_a)
      drain_acc(dloc_bb, gidx_bb, rows_bb, n_b, gsem_b)
      return n_a

    n_last = lax.fori_loop(0, (_NBLK - 2) // 2, pair, n0)
    n_tail = prep_fire(_NBLK - 1, eb_b, dloc_bb, gidx_bb, rows_bb, gsem_b,
                       eb_b)
    drain_acc(dloc_a, gidx_a, rows_a, n_last, gsem_a)
    drain_acc(dloc_bb, gidx_bb, rows_bb, n_tail, gsem_b)
    edge_copy(0, eb_b).wait()
    edge_copy(0, eb_a).wait()

    @pl.when(eseg == 1)
    def _publish():
      pltpu.sync_copy(agg_l.at[pl.ds(0, _NHP)], mrg_hbm.at[half, chunk])
    plsc.subcore_barrier()

    @pl.when(eseg == 0)
    def _merge():
      off = 0
      for sz in (_B, _B, _B, _B, _B, _B, _NHP - 6 * _B):
        pltpu.sync_copy(mrg_hbm.at[half, chunk, pl.ds(off, sz)],
                        rows_a.at[pl.ds(0, sz)])
        def mg(i, carry, off=off):
          r = off + i
          agg_l[r] = jnp.maximum(agg_l[r], rows_a[i])
          return carry
        lax.fori_loop(0, sz, mg, 0)
        off += sz

    @pl.when(eseg == 0)
    def _store():
      pltpu.sync_copy(agg_l.at[pl.ds(0, _NH)],
                      out_hbm.at[chunk, pl.ds(lo, _NH)])

  return k(hp_flat, edge_index)[0]


def _layer(h, edge_index, Wp, bp, Ws, Wn, bias, g, be):
  hp = _tc_pool(h, Wp, bp)
  agg3 = _sc_segment_max(hp.reshape(_NP * _NCH, _L), edge_index)
  agg = agg3.transpose(1, 0, 2).reshape(_N, _D)
  return _tc_combine(h, agg, Ws, Wn, bias, g, be)


def kernel(h, edge_index,
           W_pool0, b_pool0, W_self0, W_neigh0, bias0, ln_g0, ln_b0,
           W_pool1, b_pool1, W_self1, W_neigh1, bias1, ln_g1, ln_b1):
  h = _layer(h, edge_index, W_pool0, b_pool0, W_self0, W_neigh0, bias0, ln_g0, ln_b0)
  h = _layer(h, edge_index, W_pool1, b_pool1, W_self1, W_neigh1, bias1, ln_g1, ln_b1)
  return h


# speedup vs baseline: 2.4504x; 1.0103x over previous
import functools

import jax
import jax.numpy as jnp
from jax import lax
from jax.experimental import pallas as pl
from jax.experimental.pallas import tpu as pltpu
from jax.experimental.pallas import tpu_sc as plsc

_N = 10000
_E = 320000
_D = 128
_EPS = 1e-5
_L = 16
_NCH = _D // _L
_NH = _N // 2
_NHP = 5008
_EH = _E // 2
_B = 800
_NBLK = _EH // _B
_GCH = 80

_NP = 10016
_DUMMY_G = _N * _NCH
_DUMMY_A = _NH

_ROWS_TC = 2504
_ROWS_TC2 = 2000


def _tc_pool(h, W, b):
  def body(h_ref, w_ref, b_ref, o_ref):
    i = pl.program_id(0)
    y = lax.dot_general(h_ref[...], w_ref[...], (((1,), (1,)), ((), ())),
                        preferred_element_type=jnp.float32)
    y = y + b_ref[...]
    row = i * _ROWS_TC + lax.broadcasted_iota(jnp.int32, y.shape, 0)
    o_ref[...] = jnp.where(row < _N, jnp.maximum(y, 0.0), 0.0)

  return pl.pallas_call(
      body,
      out_shape=jax.ShapeDtypeStruct((_NP, _D), jnp.float32),
      grid=(_NP // _ROWS_TC,),
      in_specs=[
          pl.BlockSpec((_ROWS_TC, _D), lambda i: (i, 0)),
          pl.BlockSpec((_D, _D), lambda i: (0, 0)),
          pl.BlockSpec((1, _D), lambda i: (0, 0)),
      ],
      out_specs=pl.BlockSpec((_ROWS_TC, _D), lambda i: (i, 0)),
  )(h, W, b.reshape(1, _D))


def _tc_combine(h, agg, Ws, Wn, bias, g, be):
  def body(h_ref, a_ref, ws_ref, wn_ref, b_ref, g_ref, e_ref, o_ref):
    x = lax.dot_general(h_ref[...], ws_ref[...], (((1,), (1,)), ((), ())),
                        preferred_element_type=jnp.float32)
    x = x + lax.dot_general(a_ref[...], wn_ref[...], (((1,), (1,)), ((), ())),
                            preferred_element_type=jnp.float32)
    x = x + b_ref[...]
    mu = jnp.mean(x, axis=1, keepdims=True)
    xc = x - mu
    var = jnp.mean(xc * xc, axis=1, keepdims=True)
    y = xc * lax.rsqrt(var + _EPS) * g_ref[...] + e_ref[...]
    o_ref[...] = jnp.maximum(y, 0.0)

  return pl.pallas_call(
      body,
      out_shape=jax.ShapeDtypeStruct((_N, _D), jnp.float32),
      grid=(_N // _ROWS_TC2,),
      in_specs=[
          pl.BlockSpec((_ROWS_TC2, _D), lambda i: (i, 0)),
          pl.BlockSpec((_ROWS_TC2, _D), lambda i: (i, 0)),
          pl.BlockSpec((_D, _D), lambda i: (0, 0)),
          pl.BlockSpec((_D, _D), lambda i: (0, 0)),
          pl.BlockSpec((1, _D), lambda i: (0, 0)),
          pl.BlockSpec((1, _D), lambda i: (0, 0)),
          pl.BlockSpec((1, _D), lambda i: (0, 0)),
      ],
      out_specs=pl.BlockSpec((_ROWS_TC2, _D), lambda i: (i, 0)),
  )(h, agg, Ws, Wn, bias.reshape(1, _D), g.reshape(1, _D), be.reshape(1, _D))


def _sc_segment_max(hp_flat, edge_index):
  mesh = plsc.VectorSubcoreMesh(core_axis_name="c", subcore_axis_name="s")

  @functools.partial(
      pl.kernel,
      out_type=(jax.ShapeDtypeStruct((_NCH, _N, _L), jnp.float32),
                jax.ShapeDtypeStruct((2, _NCH, _NHP, _L), jnp.float32)),
      mesh=mesh,
      compiler_params=pltpu.CompilerParams(use_tc_tiling_on_sc=False,
                                           needs_layout_passes=False),
      scratch_types=[
          pltpu.VMEM((_NHP, _L), jnp.float32),
          pltpu.VMEM((2, _B), jnp.int32),
          pltpu.VMEM((2, _B), jnp.int32),
          pltpu.VMEM((_B + _L,), jnp.int32),
          pltpu.VMEM((_B,), jnp.int32),
          pltpu.VMEM((_B, _L), jnp.float32),
          pltpu.VMEM((_B + _L,), jnp.int32),
          pltpu.VMEM((_B,), jnp.int32),
          pltpu.VMEM((_B, _L), jnp.float32),
          pltpu.SemaphoreType.DMA,
          pltpu.SemaphoreType.DMA,
          pltpu.SemaphoreType.DMA,
      ],
  )
  def k(hp_hbm, ei_hbm, out_hbm, mrg_hbm,
        agg_l, eb_a, eb_b, dloc_a, gidx_a, rows_a,
        dloc_bb, gidx_bb, rows_bb, gsem_a, gsem_b, esem):
    cid = lax.axis_index("c")
    sid = lax.axis_index("s")
    chunk = sid % _NCH
    eseg = sid // _NCH
    half = cid
    lo = half * _NH
    ebase = eseg * _EH
    lanes = lax.iota(jnp.int32, _L)

    zrow = jnp.zeros((_L,), jnp.float32)
    def zero_agg(i, carry):
      agg_l[i] = zrow
      return carry
    lax.fori_loop(0, _NHP, zero_agg, 0)

    zidx = jnp.zeros((_L,), jnp.int32)
    def mk_zero_gidx(gidxb):
      def zero_gidx(i, carry):
        gidxb[pl.ds(i * _L, _L)] = zidx
        return carry
      return zero_gidx
    lax.fori_loop(0, _B // _L, mk_zero_gidx(gidx_a), 0)
    lax.fori_loop(0, _B // _L, mk_zero_gidx(gidx_bb), 0)

    def edge_copy(b, ebuf):
      bb = lax.rem(b, _NBLK)
      base = ebase + bb * _B
      return pltpu.make_async_copy(
          ei_hbm.at[:, pl.ds(base, _B)], ebuf, esem)

    def prep_fire(b, ebuf, dlocb, gidxb, rowsb, gsem, nextbuf):
      edge_copy(b, ebuf).wait()

      def prep(i, nvec):
        d = ebuf[1, pl.ds(i * _L, _L)]
        s2 = ebuf[0, pl.ds(i * _L, _L)]
        m = (d >= lo) & (d < lo + _NH)
        cs = plsc.cumsum(jnp.where(m, 1, 0))
        tgt = nvec + cs - 1
        plsc.store_scatter(dlocb, [tgt], d - lo, mask=m)
        plsc.store_scatter(gidxb, [tgt], s2 * _NCH + chunk, mask=m)
        return nvec + plsc.all_reduce_population_count(m)
      nvec = lax.fori_loop(0, _B // _L, prep,
                           jnp.zeros((_L,), jnp.int32), unroll=2)
      plsc.store_scatter(dlocb, [nvec + lanes],
                         jnp.full((_L,), _DUMMY_A, jnp.int32))
      edge_copy(b + 2, nextbuf).start()
      nsc = nvec[0]

      nch = (nsc + _GCH - 1) // _GCH
      def fire(j, carry2):
        pltpu.make_async_copy(
            hp_hbm.at[gidxb.at[pl.ds(j * _GCH, _GCH)]],
            rowsb.at[pl.ds(j * _GCH, _GCH)],
            gsem).start()
        return carry2
      lax.fori_loop(0, nch, fire, 0)
      return nsc

    def drain_acc(dlocb, gidxb, rowsb, nsc, gsem):
      nch = (nsc + _GCH - 1) // _GCH
      def drain(j, carry2):
        pltpu.make_async_copy(
            hp_hbm.at[gidxb.at[pl.ds(j * _GCH, _GCH)]],
            rowsb.at[pl.ds(j * _GCH, _GCH)],
            gsem).wait()
        return carry2
      lax.fori_loop(0, nch, drain, 0)

      ngrp = (nsc + _L - 1) // _L
      def accg(gi, carry2):
        dvec = dlocb[pl.ds(gi * _L, _L)]
        erow = gi * _L + lanes
        for kk in range(_L):
          col = (lanes + kk) & (_L - 1)
          a = plsc.load_gather(agg_l, [dvec, col])
          r = plsc.load_gather(rowsb, [erow, col])
          plsc.store_scatter(agg_l, [dvec, col], jnp.maximum(a, r))
        return carry2
      lax.fori_loop(0, ngrp, accg, 0)

    edge_copy(0, eb_a).start()
    edge_copy(1, eb_b).start()
    n0 = prep_fire(0, eb_a, dloc_a, gidx_a, rows_a, gsem_a, eb_a)

    def pair(p, n_prev):
      b1 = 2 * p + 1
      n_b = prep_fire(b1, eb_b, dloc_bb, gidx_bb, rows_bb, gsem_b, eb_b)
      drain_acc(dloc_a, gidx_a, rows_a, n_prev, gsem_a)
      n_a = prep_fire(b1 + 1, eb_a, dloc_a, gidx_a, rows_a, gsem_a, eb_a)
      drain_acc(dloc_bb, gidx_bb, rows_bb, n_b, gsem_b)
      return n_a

    n_last = lax.fori_loop(0, (_NBLK - 2) // 2, pair, n0)
    n_tail = prep_fire(_NBLK - 1, eb_b, dloc_bb, gidx_bb, rows_bb, gsem_b,
                       eb_b)
    drain_acc(dloc_a, gidx_a, rows_a, n_last, gsem_a)
    drain_acc(dloc_bb, gidx_bb, rows_bb, n_tail, gsem_b)
    edge_copy(0, eb_b).wait()
    edge_copy(0, eb_a).wait()

    @pl.when(eseg == 1)
    def _publish():
      pltpu.sync_copy(agg_l.at[pl.ds(0, _NHP)], mrg_hbm.at[half, chunk])
    plsc.subcore_barrier()

    @pl.when(eseg == 0)
    def _merge():
      off = 0
      for sz in (_B, _B, _B, _B, _B, _B, _NHP - 6 * _B):
        pltpu.sync_copy(mrg_hbm.at[half, chunk, pl.ds(off, sz)],
                        rows_a.at[pl.ds(0, sz)])
        def mg(i, carry, off=off):
          r = off + i
          agg_l[r] = jnp.maximum(agg_l[r], rows_a[i])
          return carry
        lax.fori_loop(0, sz, mg, 0)
        off += sz

    @pl.when(eseg == 0)
    def _store():
      pltpu.sync_copy(agg_l.at[pl.ds(0, _NH)],
                      out_hbm.at[chunk, pl.ds(lo, _NH)])

  return k(hp_flat, edge_index)[0]


def _layer(h, edge_index, Wp, bp, Ws, Wn, bias, g, be):
  hp = _tc_pool(h, Wp, bp)
  agg3 = _sc_segment_max(hp.reshape(_NP * _NCH, _L), edge_index)
  agg = agg3.transpose(1, 0, 2).reshape(_N, _D)
  return _tc_combine(h, agg, Ws, Wn, bias, g, be)


def kernel(h, edge_index,
           W_pool0, b_pool0, W_self0, W_neigh0, bias0, ln_g0, ln_b0,
           W_pool1, b_pool1, W_self1, W_neigh1, bias1, ln_g1, ln_b1):
  h = _layer(h, edge_index, W_pool0, b_pool0, W_self0, W_neigh0, bias0, ln_g0, ln_b0)
  h = _layer(h, edge_index, W_pool1, b_pool1, W_self1, W_neigh1, bias1, ln_g1, ln_b1)
  return h
